# baseline (speedup 1.0000x reference)
import functools

import jax
import jax.numpy as jnp
from jax import lax
from jax.experimental import pallas as pl
from jax.experimental.pallas import tpu as pltpu
from jax.experimental.pallas import tpu_sc as plsc

_LANES = 16


@functools.lru_cache(maxsize=None)
def _make_pool_t(B, L, D, V):
    info = plsc.get_sparse_core_info()
    NC, NS = info.num_cores, info.num_subcores
    NW = NC * NS
    assert D % NW == 0 and B % _LANES == 0
    d_per_w = D // NW
    mesh = plsc.VectorSubcoreMesh(core_axis_name="c", subcore_axis_name="s")

    @functools.partial(
        pl.kernel,
        mesh=mesh,
        out_type=jax.ShapeDtypeStruct((D * B,), jnp.float32),
        scratch_types=[
            pltpu.VMEM((L * B,), jnp.int32),
            pltpu.VMEM((V,), jnp.float32),
            pltpu.VMEM((d_per_w * B,), jnp.float32),
        ],
        compiler_params=pltpu.CompilerParams(
            use_tc_tiling_on_sc=False, needs_layout_passes=False
        ),
    )
    def pool(xt_hbm, embt_hbm, out_hbm, xt_v, row_v, out_v):
        wid = lax.axis_index("s") * NC + lax.axis_index("c")
        pltpu.sync_copy(xt_hbm, xt_v)
        inv_l = jnp.float32(1.0 / L)
        for dl in range(d_per_w):
            d = wid * d_per_w + dl
            pltpu.sync_copy(embt_hbm.at[d], row_v)

            @plsc.parallel_loop(0, B, step=_LANES, unroll=2)
            def body(b0):
                acc = jnp.zeros((_LANES,), jnp.float32)
                for t in range(L):
                    idx = xt_v[pl.ds(t * B + b0, _LANES)]
                    acc = acc + plsc.load_gather(row_v, [idx])
                out_v[pl.ds(dl * B + b0, _LANES)] = acc * inv_l
        pltpu.sync_copy(
            out_v, out_hbm.at[pl.ds(wid * (d_per_w * B), d_per_w * B)]
        )

    return pool


@functools.lru_cache(maxsize=None)
def _make_proj_t(B, D, V, bv=2048, nbuf=3):
    nv = pl.cdiv(V, bv)
    tail = V - (nv - 1) * bv
    assert bv % 8 == 0 and tail % 8 == 0

    def proj(w_ref, m_ref, b_ref, o_hbm, obuf, sems):
        i = pl.program_id(0)
        slot = lax.rem(i, nbuf)

        @pl.when(i >= nbuf)
        def _():
            off = pl.multiple_of((i - nbuf) * bv, bv)
            pltpu.make_async_copy(
                obuf.at[slot], o_hbm.at[pl.ds(off, bv)], sems.at[slot]
            ).wait()

        obuf[slot] = (
            lax.dot_general(
                w_ref[...],
                m_ref[...],
                dimension_numbers=(((0,), (0,)), ((), ())),
                preferred_element_type=jnp.float32,
            )
            + b_ref[...][:, None]
        )

        @pl.when(i < nv - 1)
        def _():
            off = pl.multiple_of(i * bv, bv)
            pltpu.make_async_copy(
                obuf.at[slot], o_hbm.at[pl.ds(off, bv)], sems.at[slot]
            ).start()

        @pl.when(i == nv - 1)
        def _():
            last_slot = (nv - 1) % nbuf
            pltpu.make_async_copy(
                obuf.at[last_slot, pl.ds(0, tail)],
                o_hbm.at[pl.ds((nv - 1) * bv, tail)],
                sems.at[last_slot],
            ).start()
            for k in range(1, nbuf):
                s = (nv - 1 - k) % nbuf
                pltpu.make_async_copy(
                    obuf.at[s], o_hbm.at[pl.ds(0, bv)], sems.at[s]
                ).wait()
            pltpu.make_async_copy(
                obuf.at[last_slot, pl.ds(0, tail)],
                o_hbm.at[pl.ds(0, tail)],
                sems.at[last_slot],
            ).wait()

    return pl.pallas_call(
        proj,
        grid=(nv,),
        in_specs=[
            pl.BlockSpec((D, bv), lambda i: (0, i)),
            pl.BlockSpec((D, B), lambda i: (0, 0)),
            pl.BlockSpec((bv,), lambda i: (i,)),
        ],
        out_specs=pl.BlockSpec(memory_space=pl.ANY),
        out_shape=jax.ShapeDtypeStruct((V, B), jnp.float32),
        scratch_shapes=[
            pltpu.VMEM((nbuf, bv, B), jnp.float32),
            pltpu.SemaphoreType.DMA((nbuf,)),
        ],
        compiler_params=pltpu.CompilerParams(
            dimension_semantics=("arbitrary",),
        ),
    )


def kernel(x, emb, W, b):
    B, L = x.shape
    V, D = emb.shape
    xt = x.astype(jnp.int32).T.reshape(-1)
    mt = _make_pool_t(B, L, D, V)(xt, emb.T).reshape(D, B)
    ot = _make_proj_t(B, D, V)(W, mt, b)
    return ot.T


# speedup vs baseline: 1.0046x; 1.0046x over previous
import functools

import jax
import jax.numpy as jnp
from jax import lax
from jax.experimental import pallas as pl
from jax.experimental.pallas import tpu as pltpu
from jax.experimental.pallas import tpu_sc as plsc

_LANES = 16


@functools.lru_cache(maxsize=None)
def _make_pool_t(B, L, D, V):
    info = plsc.get_sparse_core_info()
    NC, NS = info.num_cores, info.num_subcores
    NW = NC * NS
    assert D % NW == 0 and B % _LANES == 0
    d_per_w = D // NW
    mesh = plsc.VectorSubcoreMesh(core_axis_name="c", subcore_axis_name="s")

    @functools.partial(
        pl.kernel,
        mesh=mesh,
        out_type=jax.ShapeDtypeStruct((D * B,), jnp.float32),
        scratch_types=[
            pltpu.VMEM((L * B,), jnp.int32),
            pltpu.VMEM((V,), jnp.float32),
            pltpu.VMEM((d_per_w * B,), jnp.float32),
            pltpu.SemaphoreType.DMA,
            pltpu.SemaphoreType.DMA,
        ],
        compiler_params=pltpu.CompilerParams(
            use_tc_tiling_on_sc=False, needs_layout_passes=False
        ),
    )
    def pool(xt_hbm, embt_hbm, out_hbm, xt_v, row_v, out_v, sem_x, sem_r):
        wid = lax.axis_index("s") * NC + lax.axis_index("c")
        cx = pltpu.async_copy(xt_hbm, xt_v, sem_x)
        cr = pltpu.async_copy(embt_hbm.at[wid * d_per_w], row_v, sem_r)
        cx.wait()
        cr.wait()
        inv_l = jnp.float32(1.0 / L)
        for dl in range(d_per_w):
            d = wid * d_per_w + dl
            if dl > 0:
                pltpu.sync_copy(embt_hbm.at[d], row_v)

            @plsc.parallel_loop(0, B, step=_LANES, unroll=2)
            def body(b0):
                acc = jnp.zeros((_LANES,), jnp.float32)
                for t in range(L):
                    idx = xt_v[pl.ds(t * B + b0, _LANES)]
                    acc = acc + plsc.load_gather(row_v, [idx])
                out_v[pl.ds(dl * B + b0, _LANES)] = acc * inv_l
        pltpu.sync_copy(
            out_v, out_hbm.at[pl.ds(wid * (d_per_w * B), d_per_w * B)]
        )

    return pool


@functools.lru_cache(maxsize=None)
def _make_proj_t(B, D, V, bv=4096):
    nv = pl.cdiv(V, bv)

    def proj(w_ref, m_ref, b_ref, o_ref):
        o_ref[...] = (
            lax.dot_general(
                w_ref[...],
                m_ref[...],
                dimension_numbers=(((0,), (0,)), ((), ())),
                preferred_element_type=jnp.float32,
            )
            + b_ref[...][:, None]
        )

    return pl.pallas_call(
        proj,
        grid=(nv,),
        in_specs=[
            pl.BlockSpec((D, bv), lambda i: (0, i)),
            pl.BlockSpec((D, B), lambda i: (0, 0)),
            pl.BlockSpec((bv,), lambda i: (i,)),
        ],
        out_specs=pl.BlockSpec((bv, B), lambda i: (i, 0)),
        out_shape=jax.ShapeDtypeStruct((V, B), jnp.float32),
        compiler_params=pltpu.CompilerParams(
            dimension_semantics=("parallel",),
            fuse_transposed_lhs_in_matmul=True,
        ),
    )


def kernel(x, emb, W, b):
    B, L = x.shape
    V, D = emb.shape
    xt = x.astype(jnp.int32).T.reshape(-1)
    mt = _make_pool_t(B, L, D, V)(xt, emb.T).reshape(D, B)
    ot = _make_proj_t(B, D, V)(W, mt, b)
    return ot.T
